# baseline (speedup 1.0000x reference)
import functools

import jax
import jax.numpy as jnp
from jax import lax
from jax.experimental import pallas as pl
from jax.experimental.pallas import tpu as pltpu
from jax.experimental.pallas import tpu_sc as plsc

_T = 32768
_H = 768
_E = 8
_BLK = 2048

_CHUNKS = (20480, 12288)
_TB = _BLK // 128

_NC, _NS, _L = 2, 16, 16
_NW = _NC * _NS


def _gate_body(x_ref, w_ref, o_ref):
    lg = lax.dot_general(
        w_ref[...], x_ref[...],
        (((1,), (1,)), ((), ())),
        preferred_element_type=jnp.float32,
    )
    for b in range(_TB):
        o_ref[b] = lg[:, 128 * b:128 * (b + 1)]


def _gate_logits(x, W, start_tok, tch):
    steps = tch // _BLK
    step0 = start_tok // _BLK
    return pl.pallas_call(
        _gate_body,
        grid=(steps,),
        in_specs=[
            pl.BlockSpec((_BLK, _H), lambda i: (step0 + i, 0)),
            pl.BlockSpec((_E, _H), lambda i: (0, 0)),
        ],
        out_specs=pl.BlockSpec((_TB, _E, 128), lambda i: (i, 0, 0)),
        out_shape=jax.ShapeDtypeStruct((tch // 128, _E, 128), jnp.float32),
    )(x, W)


def _route_body(tpw, bpw,
                lg_hbm, w1_hbm, w2_hbm, i1_hbm, i2_hbm,
                l_v, ow1_v, ow2_v, oi1_v, oi2_v):
    wid = lax.axis_index("s") * _NC + lax.axis_index("c")
    base = wid * tpw
    pltpu.sync_copy(lg_hbm.at[pl.ds(wid * bpw, bpw)], l_v)

    def body(sub, carry):
        off = sub * _L
        for b in range(bpw):
            s = [l_v[b, e, pl.ds(off, _L)] for e in range(_E)]
            m1 = s[0]
            i1 = jnp.zeros((_L,), jnp.int32)
            for e in range(1, _E):
                gt = s[e] > m1
                m1 = jnp.where(gt, s[e], m1)
                i1 = jnp.where(gt, e, i1)
            neg = jnp.float32(-3.0e38)
            m2 = jnp.full((_L,), neg, jnp.float32)
            i2 = jnp.zeros((_L,), jnp.int32)
            for e in range(_E):
                cand = jnp.where(i1 == e, neg, s[e])
                gt = cand > m2
                m2 = jnp.where(gt, cand, m2)
                i2 = jnp.where(gt, e, i2)
            t = jnp.exp(m2 - m1)
            ssq = 1.0 + t * t
            bi = lax.bitcast_convert_type(ssq, jnp.int32)
            bi = 0x5F3759DF - (bi >> 1)
            y = lax.bitcast_convert_type(bi, jnp.float32)
            for _ in range(3):
                y = y * (1.5 - 0.5 * ssq * y * y)
            col = b * 128 + off
            ow1_v[pl.ds(col, _L)] = 2.0 * y
            ow2_v[pl.ds(col, _L)] = 2.0 * t * y
            oi1_v[pl.ds(col, _L)] = i1
            oi2_v[pl.ds(col, _L)] = i2
        return carry

    lax.fori_loop(0, 128 // _L, body, 0)
    pltpu.sync_copy(ow1_v, w1_hbm.at[pl.ds(base, tpw)])
    pltpu.sync_copy(ow2_v, w2_hbm.at[pl.ds(base, tpw)])
    pltpu.sync_copy(oi1_v, i1_hbm.at[pl.ds(base, tpw)])
    pltpu.sync_copy(oi2_v, i2_hbm.at[pl.ds(base, tpw)])


@functools.cache
def _route(tch):
    tpw = tch // _NW
    bpw = tpw // 128
    return pl.kernel(
        functools.partial(_route_body, tpw, bpw),
        out_type=[
            jax.ShapeDtypeStruct((tch,), jnp.float32),
            jax.ShapeDtypeStruct((tch,), jnp.float32),
            jax.ShapeDtypeStruct((tch,), jnp.int32),
            jax.ShapeDtypeStruct((tch,), jnp.int32),
        ],
        mesh=plsc.VectorSubcoreMesh(
            core_axis_name="c", subcore_axis_name="s",
            num_cores=_NC, num_subcores=_NS,
        ),
        scratch_types=[
            pltpu.VMEM((bpw, _E, 128), jnp.float32),
            pltpu.VMEM((tpw,), jnp.float32),
            pltpu.VMEM((tpw,), jnp.float32),
            pltpu.VMEM((tpw,), jnp.int32),
            pltpu.VMEM((tpw,), jnp.int32),
        ],
        compiler_params=pltpu.CompilerParams(
            needs_layout_passes=False, use_tc_tiling_on_sc=False),
    )


def kernel(x, W):
    ews, eis = [], []
    start = 0
    for tch in _CHUNKS:
        w1, w2, i1, i2 = _route(tch)(_gate_logits(x, W, start, tch))
        ews.append(jnp.stack([w1, w2], axis=-1))
        eis.append(jnp.stack([i1, i2], axis=-1))
        start += tch
    ew = jnp.concatenate(ews) if len(_CHUNKS) > 1 else ews[0]
    ei = jnp.concatenate(eis) if len(_CHUNKS) > 1 else eis[0]
    return ew, ei


# speedup vs baseline: 1.0001x; 1.0001x over previous
import functools

import jax
import jax.numpy as jnp
from jax import lax
from jax.experimental import pallas as pl
from jax.experimental.pallas import tpu as pltpu
from jax.experimental.pallas import tpu_sc as plsc

_T = 32768
_H = 768
_E = 8
_BLK = 2048

_CHUNKS = (32768,)
_TB = _BLK // 128

_NC, _NS, _L = 2, 16, 16
_NW = _NC * _NS


def _gate_body(x_ref, w_ref, o_ref):
    lg = lax.dot_general(
        w_ref[...], x_ref[...],
        (((1,), (1,)), ((), ())),
        preferred_element_type=jnp.float32,
    )
    for b in range(_TB):
        o_ref[b] = lg[:, 128 * b:128 * (b + 1)]


def _gate_logits(x, W, start_tok, tch):
    steps = tch // _BLK
    step0 = start_tok // _BLK
    return pl.pallas_call(
        _gate_body,
        grid=(steps,),
        in_specs=[
            pl.BlockSpec((_BLK, _H), lambda i: (step0 + i, 0)),
            pl.BlockSpec((_E, _H), lambda i: (0, 0)),
        ],
        out_specs=pl.BlockSpec((_TB, _E, 128), lambda i: (i, 0, 0)),
        out_shape=jax.ShapeDtypeStruct((tch // 128, _E, 128), jnp.float32),
    )(x, W)


def _route_body(tpw, bpw,
                lg_hbm, w1_hbm, w2_hbm, i1_hbm, i2_hbm,
                l_v, ow1_v, ow2_v, oi1_v, oi2_v):
    wid = lax.axis_index("s") * _NC + lax.axis_index("c")
    base = wid * tpw
    pltpu.sync_copy(lg_hbm.at[pl.ds(wid * bpw, bpw)], l_v)

    def body(sub, carry):
        off = sub * _L
        for b in range(bpw):
            s = [l_v[b, e, pl.ds(off, _L)] for e in range(_E)]
            m1 = s[0]
            i1 = jnp.zeros((_L,), jnp.int32)
            for e in range(1, _E):
                gt = s[e] > m1
                m1 = jnp.where(gt, s[e], m1)
                i1 = jnp.where(gt, e, i1)
            neg = jnp.float32(-3.0e38)
            m2 = jnp.full((_L,), neg, jnp.float32)
            i2 = jnp.zeros((_L,), jnp.int32)
            for e in range(_E):
                cand = jnp.where(i1 == e, neg, s[e])
                gt = cand > m2
                m2 = jnp.where(gt, cand, m2)
                i2 = jnp.where(gt, e, i2)
            t = jnp.exp(m2 - m1)
            ssq = 1.0 + t * t
            bi = lax.bitcast_convert_type(ssq, jnp.int32)
            bi = 0x5F3759DF - (bi >> 1)
            y = lax.bitcast_convert_type(bi, jnp.float32)
            for _ in range(3):
                y = y * (1.5 - 0.5 * ssq * y * y)
            col = b * 128 + off
            ow1_v[pl.ds(col, _L)] = 2.0 * y
            ow2_v[pl.ds(col, _L)] = 2.0 * t * y
            oi1_v[pl.ds(col, _L)] = i1
            oi2_v[pl.ds(col, _L)] = i2
        return carry

    lax.fori_loop(0, 128 // _L, body, 0)
    pltpu.sync_copy(ow1_v, w1_hbm.at[pl.ds(base, tpw)])
    pltpu.sync_copy(ow2_v, w2_hbm.at[pl.ds(base, tpw)])
    pltpu.sync_copy(oi1_v, i1_hbm.at[pl.ds(base, tpw)])
    pltpu.sync_copy(oi2_v, i2_hbm.at[pl.ds(base, tpw)])


@functools.cache
def _route(tch):
    tpw = tch // _NW
    bpw = tpw // 128
    return pl.kernel(
        functools.partial(_route_body, tpw, bpw),
        out_type=[
            jax.ShapeDtypeStruct((tch,), jnp.float32),
            jax.ShapeDtypeStruct((tch,), jnp.float32),
            jax.ShapeDtypeStruct((tch,), jnp.int32),
            jax.ShapeDtypeStruct((tch,), jnp.int32),
        ],
        mesh=plsc.VectorSubcoreMesh(
            core_axis_name="c", subcore_axis_name="s",
            num_cores=_NC, num_subcores=_NS,
        ),
        scratch_types=[
            pltpu.VMEM((bpw, _E, 128), jnp.float32),
            pltpu.VMEM((tpw,), jnp.float32),
            pltpu.VMEM((tpw,), jnp.float32),
            pltpu.VMEM((tpw,), jnp.int32),
            pltpu.VMEM((tpw,), jnp.int32),
        ],
        compiler_params=pltpu.CompilerParams(
            needs_layout_passes=False, use_tc_tiling_on_sc=False),
    )


def kernel(x, W):
    ews, eis = [], []
    start = 0
    for tch in _CHUNKS:
        w1, w2, i1, i2 = _route(tch)(_gate_logits(x, W, start, tch))
        ews.append(jnp.stack([w1, w2], axis=-1))
        eis.append(jnp.stack([i1, i2], axis=-1))
        start += tch
    ew = jnp.concatenate(ews) if len(_CHUNKS) > 1 else ews[0]
    ei = jnp.concatenate(eis) if len(_CHUNKS) > 1 else eis[0]
    return ew, ei


# speedup vs baseline: 1.0413x; 1.0412x over previous
import functools

import jax
import jax.numpy as jnp
from jax import lax
from jax.experimental import pallas as pl
from jax.experimental.pallas import tpu as pltpu
from jax.experimental.pallas import tpu_sc as plsc

_T = 32768
_H = 768
_E = 8
_BLK = 4096

_CHUNKS = (32768,)
_TB = _BLK // 128

_NC, _NS, _L = 2, 16, 16
_NW = _NC * _NS


def _gate_body(x_ref, w_ref, o_ref):
    lg = lax.dot_general(
        w_ref[...], x_ref[...],
        (((1,), (1,)), ((), ())),
        preferred_element_type=jnp.float32,
    )
    for b in range(_TB):
        o_ref[b] = lg[:, 128 * b:128 * (b + 1)]


def _gate_logits(x, W, start_tok, tch):
    steps = tch // _BLK
    step0 = start_tok // _BLK
    return pl.pallas_call(
        _gate_body,
        grid=(steps,),
        in_specs=[
            pl.BlockSpec((_BLK, _H), lambda i: (step0 + i, 0)),
            pl.BlockSpec((_E, _H), lambda i: (0, 0)),
        ],
        out_specs=pl.BlockSpec((_TB, _E, 128), lambda i: (i, 0, 0)),
        out_shape=jax.ShapeDtypeStruct((tch // 128, _E, 128), jnp.float32),
    )(x, W)


def _route_body(tpw, bpw,
                lg_hbm, w1_hbm, w2_hbm, i1_hbm, i2_hbm,
                l_v, ow1_v, ow2_v, oi1_v, oi2_v):
    wid = lax.axis_index("s") * _NC + lax.axis_index("c")
    base = wid * tpw
    pltpu.sync_copy(lg_hbm.at[pl.ds(wid * bpw, bpw)], l_v)

    def body(sub, carry):
        off = sub * _L
        for b in range(bpw):
            s = [l_v[b, e, pl.ds(off, _L)] for e in range(_E)]
            m1 = s[0]
            i1 = jnp.zeros((_L,), jnp.int32)
            for e in range(1, _E):
                gt = s[e] > m1
                m1 = jnp.where(gt, s[e], m1)
                i1 = jnp.where(gt, e, i1)
            neg = jnp.float32(-3.0e38)
            m2 = jnp.full((_L,), neg, jnp.float32)
            i2 = jnp.zeros((_L,), jnp.int32)
            for e in range(_E):
                cand = jnp.where(i1 == e, neg, s[e])
                gt = cand > m2
                m2 = jnp.where(gt, cand, m2)
                i2 = jnp.where(gt, e, i2)
            t = jnp.exp(m2 - m1)
            ssq = 1.0 + t * t
            bi = lax.bitcast_convert_type(ssq, jnp.int32)
            bi = 0x5F3759DF - (bi >> 1)
            y = lax.bitcast_convert_type(bi, jnp.float32)
            for _ in range(3):
                y = y * (1.5 - 0.5 * ssq * y * y)
            col = b * 128 + off
            ow1_v[pl.ds(col, _L)] = 2.0 * y
            ow2_v[pl.ds(col, _L)] = 2.0 * t * y
            oi1_v[pl.ds(col, _L)] = i1
            oi2_v[pl.ds(col, _L)] = i2
        return carry

    lax.fori_loop(0, 128 // _L, body, 0)
    pltpu.sync_copy(ow1_v, w1_hbm.at[pl.ds(base, tpw)])
    pltpu.sync_copy(ow2_v, w2_hbm.at[pl.ds(base, tpw)])
    pltpu.sync_copy(oi1_v, i1_hbm.at[pl.ds(base, tpw)])
    pltpu.sync_copy(oi2_v, i2_hbm.at[pl.ds(base, tpw)])


@functools.cache
def _route(tch):
    tpw = tch // _NW
    bpw = tpw // 128
    return pl.kernel(
        functools.partial(_route_body, tpw, bpw),
        out_type=[
            jax.ShapeDtypeStruct((tch,), jnp.float32),
            jax.ShapeDtypeStruct((tch,), jnp.float32),
            jax.ShapeDtypeStruct((tch,), jnp.int32),
            jax.ShapeDtypeStruct((tch,), jnp.int32),
        ],
        mesh=plsc.VectorSubcoreMesh(
            core_axis_name="c", subcore_axis_name="s",
            num_cores=_NC, num_subcores=_NS,
        ),
        scratch_types=[
            pltpu.VMEM((bpw, _E, 128), jnp.float32),
            pltpu.VMEM((tpw,), jnp.float32),
            pltpu.VMEM((tpw,), jnp.float32),
            pltpu.VMEM((tpw,), jnp.int32),
            pltpu.VMEM((tpw,), jnp.int32),
        ],
        compiler_params=pltpu.CompilerParams(
            needs_layout_passes=False, use_tc_tiling_on_sc=False),
    )


def kernel(x, W):
    ews, eis = [], []
    start = 0
    for tch in _CHUNKS:
        w1, w2, i1, i2 = _route(tch)(_gate_logits(x, W, start, tch))
        ews.append(jnp.stack([w1, w2], axis=-1))
        eis.append(jnp.stack([i1, i2], axis=-1))
        start += tch
    ew = jnp.concatenate(ews) if len(_CHUNKS) > 1 else ews[0]
    ei = jnp.concatenate(eis) if len(_CHUNKS) > 1 else eis[0]
    return ew, ei
